# baseline (speedup 1.0000x reference)
import jax
import jax.numpy as jnp
from jax import lax
from jax.experimental import pallas as pl
from jax.experimental.pallas import tpu as pltpu
from jax.experimental.pallas import tpu_sc as plsc

BATCH = 8192
D = 2048
NC = 2
NS = 16
NW = NC * NS
L = 16
ROWS_PER_W = BATCH // NW
CHUNK = 2
NBUF = 8
NCHUNKS = ROWS_PER_W // CHUNK
NQUADS = NCHUNKS // NBUF
GROUPS = D // L
UNROLL = 8


def _permute_body(in_hbm, perm_hbm, out_hbm, perm_v, *bufs_and_sems):
    in_bufs = bufs_and_sems[0:NBUF]
    out_bufs = bufs_and_sems[NBUF:2 * NBUF]
    sem_is = bufs_and_sems[2 * NBUF:3 * NBUF]
    sem_os = bufs_and_sems[3 * NBUF:4 * NBUF]

    wid = lax.axis_index("s") * NC + lax.axis_index("c")
    base_row = wid * ROWS_PER_W

    def start_in(c, buf, sem):
        pltpu.async_copy(
            in_hbm.at[pl.ds(base_row + c * CHUNK, CHUNK), :], buf, sem)

    def start_out(c, buf, sem):
        pltpu.async_copy(
            buf, out_hbm.at[pl.ds(base_row + c * CHUNK, CHUNK), :], sem)

    def wait(buf, sem):
        pltpu.make_async_copy(in_hbm.at[pl.ds(0, CHUNK), :], buf, sem).wait()

    def gather_chunk(src, dst):
        rvecs = [jnp.full((L,), r, dtype=jnp.int32) for r in range(CHUNK)]

        @plsc.parallel_loop(0, GROUPS, unroll=UNROLL)
        def _(g):
            gl = g * L
            idx = perm_v[pl.ds(gl, L)]
            for r in range(CHUNK):
                vals = plsc.load_gather(src, [rvecs[r], idx])
                dst[r, pl.ds(gl, L)] = vals

    for b in range(NBUF):
        start_in(b, in_bufs[b], sem_is[b])
    pltpu.sync_copy(perm_hbm, perm_v)

    def quad_body(q, _):
        c0 = q * NBUF
        for b in range(NBUF):
            c = c0 + b
            wait(in_bufs[b], sem_is[b])

            @pl.when(q > 0)
            def _():
                wait(out_bufs[b], sem_os[b])
            gather_chunk(in_bufs[b], out_bufs[b])
            start_out(c, out_bufs[b], sem_os[b])

            @pl.when(q < NQUADS - 1)
            def _():
                start_in(c + NBUF, in_bufs[b], sem_is[b])
        return 0

    lax.fori_loop(0, NQUADS, quad_body, 0)

    for b in range(NBUF):
        wait(out_bufs[b], sem_os[b])


@jax.jit
def _permute(inputs, perm):
    mesh = plsc.VectorSubcoreMesh(core_axis_name="c", subcore_axis_name="s")
    return pl.kernel(
        _permute_body,
        mesh=mesh,
        out_type=jax.ShapeDtypeStruct((BATCH, D), jnp.float32),
        scratch_types=(
            [pltpu.VMEM((D,), jnp.int32)]
            + [pltpu.VMEM((CHUNK, D), jnp.float32)] * (2 * NBUF)
            + [pltpu.SemaphoreType.DMA] * (2 * NBUF)
        ),
        compiler_params=pltpu.CompilerParams(needs_layout_passes=False),
    )(inputs, perm)


def kernel(inputs, perm):
    out = _permute(inputs, perm.astype(jnp.int32))
    logdet = jnp.zeros(inputs.shape[:1], dtype=inputs.dtype)
    return (out, logdet)


# speedup vs baseline: 1.0155x; 1.0155x over previous
import jax
import jax.numpy as jnp
from jax import lax
from jax.experimental import pallas as pl
from jax.experimental.pallas import tpu as pltpu
from jax.experimental.pallas import tpu_sc as plsc

BATCH = 8192
D = 2048
NC = 2
NS = 16
NW = NC * NS
L = 16
ROWS_PER_W = BATCH // NW
CHUNK = 4
NBUF_IN = 8
NBUF_OUT = 4
NCHUNKS = ROWS_PER_W // CHUNK
NROUNDS = NCHUNKS // NBUF_IN
GROUPS = D // L
UNROLL = 8


def _permute_body(in_hbm, perm_hbm, out_hbm, perm_v, *bufs_and_sems):
    in_bufs = bufs_and_sems[0:NBUF_IN]
    out_bufs = bufs_and_sems[NBUF_IN:NBUF_IN + NBUF_OUT]
    sem_is = bufs_and_sems[NBUF_IN + NBUF_OUT:2 * NBUF_IN + NBUF_OUT]
    sem_os = bufs_and_sems[2 * NBUF_IN + NBUF_OUT:]

    wid = lax.axis_index("s") * NC + lax.axis_index("c")
    base_row = wid * ROWS_PER_W

    def start_in(c, buf, sem):
        pltpu.async_copy(
            in_hbm.at[pl.ds(base_row + c * CHUNK, CHUNK), :], buf, sem)

    def start_out(c, buf, sem):
        pltpu.async_copy(
            buf, out_hbm.at[pl.ds(base_row + c * CHUNK, CHUNK), :], sem)

    def wait(buf, sem):
        pltpu.make_async_copy(in_hbm.at[pl.ds(0, CHUNK), :], buf, sem).wait()

    def gather_chunk(src, dst):
        rvecs = [jnp.full((L,), r, dtype=jnp.int32) for r in range(CHUNK)]

        @plsc.parallel_loop(0, GROUPS, unroll=UNROLL)
        def _(g):
            gl = g * L
            idx = perm_v[pl.ds(gl, L)]
            for r in range(CHUNK):
                vals = plsc.load_gather(src, [rvecs[r], idx])
                dst[r, pl.ds(gl, L)] = vals

    for b in range(NBUF_IN):
        start_in(b, in_bufs[b], sem_is[b])
    pltpu.sync_copy(perm_hbm, perm_v)

    def round_body(q, _):
        c0 = q * NBUF_IN
        for b in range(NBUF_IN):
            c = c0 + b
            ob = b % NBUF_OUT
            wait(in_bufs[b], sem_is[b])

            @pl.when(c >= NBUF_OUT)
            def _():
                wait(out_bufs[ob], sem_os[ob])
            gather_chunk(in_bufs[b], out_bufs[ob])
            start_out(c, out_bufs[ob], sem_os[ob])

            @pl.when(q < NROUNDS - 1)
            def _():
                start_in(c + NBUF_IN, in_bufs[b], sem_is[b])
        return 0

    lax.fori_loop(0, NROUNDS, round_body, 0)

    for b in range(NBUF_OUT):
        wait(out_bufs[b], sem_os[b])


@jax.jit
def _permute(inputs, perm):
    mesh = plsc.VectorSubcoreMesh(core_axis_name="c", subcore_axis_name="s")
    return pl.kernel(
        _permute_body,
        mesh=mesh,
        out_type=jax.ShapeDtypeStruct((BATCH, D), jnp.float32),
        scratch_types=(
            [pltpu.VMEM((D,), jnp.int32)]
            + [pltpu.VMEM((CHUNK, D), jnp.float32)] * (NBUF_IN + NBUF_OUT)
            + [pltpu.SemaphoreType.DMA] * (NBUF_IN + NBUF_OUT)
        ),
        compiler_params=pltpu.CompilerParams(needs_layout_passes=False),
    )(inputs, perm)


def kernel(inputs, perm):
    out = _permute(inputs, perm.astype(jnp.int32))
    logdet = jnp.zeros(inputs.shape[:1], dtype=inputs.dtype)
    return (out, logdet)
